# baseline (speedup 1.0000x reference)
import functools

import jax
import jax.numpy as jnp
from jax import lax
from jax.experimental import pallas as pl
from jax.experimental.pallas import tpu as pltpu
from jax.experimental.pallas import tpu_sc as plsc

N = 10000
E = 320000
D = 128
NC = 2
NS = 16
NW = NC * NS
N_PAD = 10240
ROWS_PT = N_PAD // NS
CHUNK = 128
NCHUNK = 80
EPW_PAD = NCHUNK * CHUNK
PADW = EPW_PAD - E // NW

_sc_mesh = plsc.VectorSubcoreMesh(core_axis_name="c", subcore_axis_name="s")


@functools.partial(
    pl.kernel,
    out_type=jax.ShapeDtypeStruct((NC, N_PAD, D), jnp.float32),
    mesh=_sc_mesh,
    scratch_types=[
        pltpu.VMEM((NCHUNK, CHUNK), jnp.int32),
        pltpu.VMEM((CHUNK, D), jnp.float32),
        pltpu.VMEM_SHARED((N_PAD, D), jnp.float32),
        pltpu.SemaphoreType.DMA,
    ],
)
def _deg_kernel(dst_hbm, ones_hbm, zeros_hbm, out_hbm, dst_v, ones_v, accum, sem):
    cid = lax.axis_index("c")
    sid = lax.axis_index("s")
    wid = sid * NC + cid
    row0 = sid * ROWS_PT
    pltpu.sync_copy(zeros_hbm.at[pl.ds(row0, ROWS_PT)], accum.at[pl.ds(row0, ROWS_PT)])
    pltpu.sync_copy(dst_hbm.at[wid], dst_v)
    pltpu.sync_copy(ones_hbm, ones_v)
    plsc.subcore_barrier()

    DEPTH = 4
    for k in range(DEPTH):
        pltpu.async_copy(ones_v, accum.at[dst_v.at[k]], sem, add=True)

    def body(i, carry):
        @pl.when(i + DEPTH < NCHUNK)
        def _():
            pltpu.async_copy(ones_v, accum.at[dst_v.at[i + DEPTH]], sem, add=True)

        pltpu.make_async_copy(ones_v, accum.at[dst_v.at[i]], sem).wait()
        return carry

    lax.fori_loop(0, NCHUNK, body, 0)
    plsc.subcore_barrier()
    pltpu.sync_copy(accum.at[pl.ds(row0, ROWS_PT)],
                    out_hbm.at[cid, pl.ds(row0, ROWS_PT)])


ACHUNK = 64
ANCH = EPW_PAD // ACHUNK
NBUF = 2


@functools.partial(
    pl.kernel,
    out_type=jax.ShapeDtypeStruct((NC, N_PAD, D), jnp.float32),
    mesh=_sc_mesh,
    scratch_types=[
        pltpu.VMEM((EPW_PAD,), jnp.int32),
        pltpu.VMEM((ANCH, ACHUNK), jnp.int32),
        pltpu.VMEM((NBUF, ACHUNK, D), jnp.float32),
        pltpu.VMEM_SHARED((N_PAD, D), jnp.float32),
        pltpu.SemaphoreType.DMA,
        pltpu.SemaphoreType.DMA,
    ],
)
def _agg_kernel(g_hbm, src_hbm, dst_hbm, zeros_hbm, out_hbm,
                src_f, dst_v, rows, accum, s0, s1):
    cid = lax.axis_index("c")
    sid = lax.axis_index("s")
    wid = sid * NC + cid
    row0 = sid * ROWS_PT
    sems = (s0, s1)
    pltpu.sync_copy(zeros_hbm.at[pl.ds(row0, ROWS_PT)], accum.at[pl.ds(row0, ROWS_PT)])
    pltpu.sync_copy(src_hbm.at[wid], src_f)
    pltpu.sync_copy(dst_hbm.at[wid], dst_v)
    plsc.subcore_barrier()

    pltpu.async_copy(g_hbm.at[src_f.at[pl.ds(0, ACHUNK)]], rows.at[0], s0)

    def pair(p, carry):
        for k in range(2):
            i = 2 * p + k
            pltpu.make_async_copy(g_hbm.at[src_f.at[pl.ds(i * ACHUNK, ACHUNK)]],
                                  rows.at[k], sems[k]).wait()
            pltpu.async_copy(rows.at[k], accum.at[dst_v.at[i]], sems[k],
                             add=True)

            @pl.when(i + 1 < ANCH)
            def _():
                @pl.when(i >= 1)
                def _():
                    pltpu.make_async_copy(rows.at[1 - k],
                                          accum.at[dst_v.at[i - 1]],
                                          sems[1 - k]).wait()

                pltpu.async_copy(
                    g_hbm.at[src_f.at[pl.ds((i + 1) * ACHUNK, ACHUNK)]],
                    rows.at[1 - k], sems[1 - k])

        return carry

    lax.fori_loop(0, ANCH // 2, pair, 0)
    for t in (ANCH - 2, ANCH - 1):
        pltpu.make_async_copy(rows.at[t % NBUF],
                              accum.at[dst_v.at[t]], sems[t % NBUF]).wait()

    plsc.subcore_barrier()
    pltpu.sync_copy(accum.at[pl.ds(row0, ROWS_PT)],
                    out_hbm.at[cid, pl.ds(row0, ROWS_PT)])


BR = 1024
GRID = N_PAD // BR

_row_spec = pl.BlockSpec((BR, D), lambda i: (i, 0))
_w_spec = pl.BlockSpec((D, D), lambda i: (0, 0))
_b_spec = pl.BlockSpec((1, D), lambda i: (0, 0))
_out_shape = jax.ShapeDtypeStruct((N_PAD, D), jnp.float32)


def _matmul_body(x_ref, w_ref, h_ref):
    h_ref[...] = jnp.dot(x_ref[...], w_ref[...],
                         preferred_element_type=jnp.float32)


_matmul = pl.pallas_call(
    _matmul_body,
    grid=(GRID,),
    in_specs=[_row_spec, _w_spec],
    out_specs=_row_spec,
    out_shape=_out_shape,
)


def _prep_body(p0_ref, p1_ref, h_ref, dis_ref, g_ref):
    deg = p0_ref[...][:, :1] + p1_ref[...][:, :1] + 1.0
    dis = jnp.broadcast_to(lax.rsqrt(deg), (BR, D))
    dis_ref[...] = dis
    g_ref[...] = h_ref[...] * dis


_prep = pl.pallas_call(
    _prep_body,
    grid=(GRID,),
    in_specs=[_row_spec, _row_spec, _row_spec],
    out_specs=[_row_spec, _row_spec],
    out_shape=[_out_shape, _out_shape],
)


def _mid_body(dis_ref, a0_ref, a1_ref, g_ref, b_ref, w_ref, out_ref):
    dis = dis_ref[...]
    h = (a0_ref[...] + a1_ref[...] + g_ref[...]) * dis + b_ref[...]
    h = jnp.maximum(h, 0.0)
    out_ref[...] = jnp.dot(h, w_ref[...], preferred_element_type=jnp.float32) * dis


_mid = pl.pallas_call(
    _mid_body,
    grid=(GRID,),
    in_specs=[_row_spec, _row_spec, _row_spec, _row_spec, _b_spec, _w_spec],
    out_specs=_row_spec,
    out_shape=_out_shape,
)


def _final_body(dis_ref, a0_ref, a1_ref, g_ref, b_ref, out_ref):
    out_ref[...] = ((a0_ref[...] + a1_ref[...] + g_ref[...]) * dis_ref[...]
                    + b_ref[...])


_final = pl.pallas_call(
    _final_body,
    grid=(GRID,),
    in_specs=[_row_spec, _row_spec, _row_spec, _row_spec, _b_spec],
    out_specs=_row_spec,
    out_shape=jax.ShapeDtypeStruct((N, D), jnp.float32),
)


def _pad_edges(idx):
    per_w = idx.reshape(NW, E // NW)
    pad = jnp.broadcast_to(N + (jnp.arange(PADW, dtype=jnp.int32) % (N_PAD - N)),
                           (NW, PADW))
    return jnp.concatenate([per_w, pad], axis=1)


def kernel(x, edge_index, W0, b0, W1, b1, W2, b2):
    src_flat = _pad_edges(edge_index[0].astype(jnp.int32))
    dst_flat = _pad_edges(edge_index[1].astype(jnp.int32))
    src_a = src_flat
    dst_a = dst_flat.reshape(NW, ANCH, ACHUNK)
    dst_d = dst_flat.reshape(NW, NCHUNK, CHUNK)
    x_pad = jnp.pad(x, ((0, N_PAD - N), (0, 0)))
    ones_rows = jnp.ones((CHUNK, D), jnp.float32)
    zerosD = jnp.zeros((N_PAD, D), jnp.float32)

    h0 = _matmul(x_pad, W0)
    parts = _deg_kernel(dst_d, ones_rows, zerosD)

    dis, g0 = _prep(parts[0], parts[1], h0)
    a0 = _agg_kernel(g0, src_a, dst_a, zerosD)
    g1 = _mid(dis, a0[0], a0[1], g0, b0.reshape(1, D), W1)
    a1 = _agg_kernel(g1, src_a, dst_a, zerosD)
    g2 = _mid(dis, a1[0], a1[1], g1, b1.reshape(1, D), W2)
    a2 = _agg_kernel(g2, src_a, dst_a, zerosD)
    return _final(dis, a2[0], a2[1], g2, b2.reshape(1, D))


# speedup vs baseline: 1.2594x; 1.2594x over previous
import functools

import jax
import jax.numpy as jnp
from jax import lax
from jax.experimental import pallas as pl
from jax.experimental.pallas import tpu as pltpu
from jax.experimental.pallas import tpu_sc as plsc

N = 10000
E = 320000
D = 128
NC = 2
NS = 16
NW = NC * NS
N_PAD = 10240
ROWS_PT = N_PAD // NS
CHUNK = 128
NCHUNK = 80
EPW_PAD = NCHUNK * CHUNK
PADW = EPW_PAD - E // NW

_sc_mesh = plsc.VectorSubcoreMesh(core_axis_name="c", subcore_axis_name="s")


@functools.partial(
    pl.kernel,
    out_type=jax.ShapeDtypeStruct((NC, N_PAD, D), jnp.float32),
    mesh=_sc_mesh,
    scratch_types=[
        pltpu.VMEM((NCHUNK, CHUNK), jnp.int32),
        pltpu.VMEM((CHUNK, D), jnp.float32),
        pltpu.VMEM_SHARED((N_PAD, D), jnp.float32),
        pltpu.SemaphoreType.DMA,
    ],
)
def _deg_kernel(dst_hbm, ones_hbm, zeros_hbm, out_hbm, dst_v, ones_v, accum, sem):
    cid = lax.axis_index("c")
    sid = lax.axis_index("s")
    wid = sid * NC + cid
    row0 = sid * ROWS_PT
    pltpu.sync_copy(zeros_hbm.at[pl.ds(row0, ROWS_PT)], accum.at[pl.ds(row0, ROWS_PT)])
    pltpu.sync_copy(dst_hbm.at[wid], dst_v)
    pltpu.sync_copy(ones_hbm, ones_v)
    plsc.subcore_barrier()

    DEPTH = 4
    for k in range(DEPTH):
        pltpu.async_copy(ones_v, accum.at[dst_v.at[k]], sem, add=True)

    def body(i, carry):
        @pl.when(i + DEPTH < NCHUNK)
        def _():
            pltpu.async_copy(ones_v, accum.at[dst_v.at[i + DEPTH]], sem, add=True)

        pltpu.make_async_copy(ones_v, accum.at[dst_v.at[i]], sem).wait()
        return carry

    lax.fori_loop(0, NCHUNK, body, 0)
    plsc.subcore_barrier()
    pltpu.sync_copy(accum.at[pl.ds(row0, ROWS_PT)],
                    out_hbm.at[cid, pl.ds(row0, ROWS_PT)])


ACHUNK = 64
ANCH = EPW_PAD // ACHUNK
AGSZ = 40
NGRP = ANCH // AGSZ
NBUF = 4


@functools.partial(
    pl.kernel,
    out_type=jax.ShapeDtypeStruct((NC, N_PAD, D), jnp.float32),
    mesh=_sc_mesh,
    scratch_types=[
        pltpu.VMEM((AGSZ, ACHUNK), jnp.int32),
        pltpu.VMEM((AGSZ, ACHUNK), jnp.int32),
        pltpu.VMEM((NBUF, ACHUNK, D), jnp.float32),
        pltpu.VMEM_SHARED((N_PAD, D), jnp.float32),
        pltpu.SemaphoreType.DMA,
        pltpu.SemaphoreType.DMA,
        pltpu.SemaphoreType.DMA,
        pltpu.SemaphoreType.DMA,
    ],
)
def _agg_kernel(g_hbm, src_hbm, dst_hbm, zeros_hbm, out_hbm,
                src_v, dst_v, rows, accum, s0, s1, s2, s3):
    cid = lax.axis_index("c")
    sid = lax.axis_index("s")
    wid = sid * NC + cid
    row0 = sid * ROWS_PT
    sems = (s0, s1, s2, s3)
    pltpu.sync_copy(zeros_hbm.at[pl.ds(row0, ROWS_PT)], accum.at[pl.ds(row0, ROWS_PT)])
    plsc.subcore_barrier()

    def quad(q, carry):
        i0 = 4 * q
        for k in range(4):
            i = i0 + k
            k2 = (k + 2) % 4
            pltpu.make_async_copy(g_hbm.at[src_v.at[i]], rows.at[k],
                                  sems[k]).wait()
            pltpu.async_copy(rows.at[k], accum.at[dst_v.at[i]], sems[k],
                             add=True)

            @pl.when(i >= 2)
            def _():
                pltpu.make_async_copy(rows.at[k2], accum.at[dst_v.at[i - 2]],
                                      sems[k2]).wait()

            @pl.when(i + 2 < AGSZ)
            def _():
                pltpu.async_copy(g_hbm.at[src_v.at[i + 2]], rows.at[k2],
                                 sems[k2])

        return carry

    for grp in range(NGRP):
        pltpu.sync_copy(src_hbm.at[wid, pl.ds(grp * AGSZ, AGSZ)], src_v)
        pltpu.sync_copy(dst_hbm.at[wid, pl.ds(grp * AGSZ, AGSZ)], dst_v)
        pltpu.async_copy(g_hbm.at[src_v.at[0]], rows.at[0], s0)
        pltpu.async_copy(g_hbm.at[src_v.at[1]], rows.at[1], s1)
        lax.fori_loop(0, AGSZ // 4, quad, 0)
        for t in (AGSZ - 2, AGSZ - 1):
            pltpu.make_async_copy(rows.at[t % NBUF],
                                  accum.at[dst_v.at[t]], sems[t % NBUF]).wait()

    plsc.subcore_barrier()
    pltpu.sync_copy(accum.at[pl.ds(row0, ROWS_PT)],
                    out_hbm.at[cid, pl.ds(row0, ROWS_PT)])


BR = 1024
GRID = N_PAD // BR

_row_spec = pl.BlockSpec((BR, D), lambda i: (i, 0))
_w_spec = pl.BlockSpec((D, D), lambda i: (0, 0))
_b_spec = pl.BlockSpec((1, D), lambda i: (0, 0))
_out_shape = jax.ShapeDtypeStruct((N_PAD, D), jnp.float32)


def _matmul_body(x_ref, w_ref, h_ref):
    h_ref[...] = jnp.dot(x_ref[...], w_ref[...],
                         preferred_element_type=jnp.float32)


_matmul = pl.pallas_call(
    _matmul_body,
    grid=(GRID,),
    in_specs=[_row_spec, _w_spec],
    out_specs=_row_spec,
    out_shape=_out_shape,
)


def _prep_body(p0_ref, p1_ref, h_ref, dis_ref, g_ref):
    deg = p0_ref[...][:, :1] + p1_ref[...][:, :1] + 1.0
    dis = jnp.broadcast_to(lax.rsqrt(deg), (BR, D))
    dis_ref[...] = dis
    g_ref[...] = h_ref[...] * dis


_prep = pl.pallas_call(
    _prep_body,
    grid=(GRID,),
    in_specs=[_row_spec, _row_spec, _row_spec],
    out_specs=[_row_spec, _row_spec],
    out_shape=[_out_shape, _out_shape],
)


def _mid_body(dis_ref, a0_ref, a1_ref, g_ref, b_ref, w_ref, out_ref):
    dis = dis_ref[...]
    h = (a0_ref[...] + a1_ref[...] + g_ref[...]) * dis + b_ref[...]
    h = jnp.maximum(h, 0.0)
    out_ref[...] = jnp.dot(h, w_ref[...], preferred_element_type=jnp.float32) * dis


_mid = pl.pallas_call(
    _mid_body,
    grid=(GRID,),
    in_specs=[_row_spec, _row_spec, _row_spec, _row_spec, _b_spec, _w_spec],
    out_specs=_row_spec,
    out_shape=_out_shape,
)


def _final_body(dis_ref, a0_ref, a1_ref, g_ref, b_ref, out_ref):
    out_ref[...] = ((a0_ref[...] + a1_ref[...] + g_ref[...]) * dis_ref[...]
                    + b_ref[...])


_final = pl.pallas_call(
    _final_body,
    grid=(GRID,),
    in_specs=[_row_spec, _row_spec, _row_spec, _row_spec, _b_spec],
    out_specs=_row_spec,
    out_shape=jax.ShapeDtypeStruct((N, D), jnp.float32),
)


def _pad_edges(idx):
    per_w = idx.reshape(NW, E // NW)
    pad = jnp.broadcast_to(N + (jnp.arange(PADW, dtype=jnp.int32) % (N_PAD - N)),
                           (NW, PADW))
    return jnp.concatenate([per_w, pad], axis=1)


def kernel(x, edge_index, W0, b0, W1, b1, W2, b2):
    src_flat = _pad_edges(edge_index[0].astype(jnp.int32))
    dst_flat = _pad_edges(edge_index[1].astype(jnp.int32))
    src_a = src_flat.reshape(NW, ANCH, ACHUNK)
    dst_a = dst_flat.reshape(NW, ANCH, ACHUNK)
    dst_d = dst_flat.reshape(NW, NCHUNK, CHUNK)
    x_pad = jnp.pad(x, ((0, N_PAD - N), (0, 0)))
    ones_rows = jnp.ones((CHUNK, D), jnp.float32)
    zerosD = jnp.zeros((N_PAD, D), jnp.float32)

    h0 = _matmul(x_pad, W0)
    parts = _deg_kernel(dst_d, ones_rows, zerosD)

    dis, g0 = _prep(parts[0], parts[1], h0)
    a0 = _agg_kernel(g0, src_a, dst_a, zerosD)
    g1 = _mid(dis, a0[0], a0[1], g0, b0.reshape(1, D), W1)
    a1 = _agg_kernel(g1, src_a, dst_a, zerosD)
    g2 = _mid(dis, a1[0], a1[1], g1, b1.reshape(1, D), W2)
    a2 = _agg_kernel(g2, src_a, dst_a, zerosD)
    return _final(dis, a2[0], a2[1], g2, b2.reshape(1, D))


# speedup vs baseline: 1.2941x; 1.0275x over previous
import functools

import jax
import jax.numpy as jnp
from jax import lax
from jax.experimental import pallas as pl
from jax.experimental.pallas import tpu as pltpu
from jax.experimental.pallas import tpu_sc as plsc

N = 10000
E = 320000
D = 128
NC = 2
NS = 16
NW = NC * NS
N_PAD = 10240
ROWS_PT = N_PAD // NS
CHUNK = 128
NCHUNK = 80
EPW_PAD = NCHUNK * CHUNK
PADW = EPW_PAD - E // NW

_sc_mesh = plsc.VectorSubcoreMesh(core_axis_name="c", subcore_axis_name="s")


@functools.partial(
    pl.kernel,
    out_type=jax.ShapeDtypeStruct((NC, N_PAD, D), jnp.float32),
    mesh=_sc_mesh,
    scratch_types=[
        pltpu.VMEM((NCHUNK, CHUNK), jnp.int32),
        pltpu.VMEM((CHUNK, D), jnp.float32),
        pltpu.VMEM_SHARED((N_PAD, D), jnp.float32),
        pltpu.SemaphoreType.DMA,
    ],
)
def _deg_kernel(dst_hbm, ones_hbm, zeros_hbm, out_hbm, dst_v, ones_v, accum, sem):
    cid = lax.axis_index("c")
    sid = lax.axis_index("s")
    wid = sid * NC + cid
    row0 = sid * ROWS_PT
    pltpu.sync_copy(zeros_hbm.at[pl.ds(row0, ROWS_PT)], accum.at[pl.ds(row0, ROWS_PT)])
    pltpu.sync_copy(dst_hbm.at[wid], dst_v)
    pltpu.sync_copy(ones_hbm, ones_v)
    plsc.subcore_barrier()

    DEPTH = 4
    for k in range(DEPTH):
        pltpu.async_copy(ones_v, accum.at[dst_v.at[k]], sem, add=True)

    def body(i, carry):
        @pl.when(i + DEPTH < NCHUNK)
        def _():
            pltpu.async_copy(ones_v, accum.at[dst_v.at[i + DEPTH]], sem, add=True)

        pltpu.make_async_copy(ones_v, accum.at[dst_v.at[i]], sem).wait()
        return carry

    lax.fori_loop(0, NCHUNK, body, 0)
    plsc.subcore_barrier()
    pltpu.sync_copy(accum.at[pl.ds(row0, ROWS_PT)],
                    out_hbm.at[cid, pl.ds(row0, ROWS_PT)])


ACHUNK = 80
ANCH = EPW_PAD // ACHUNK
AGSZ = 32
NGRP = ANCH // AGSZ
NBUF = 4


@functools.partial(
    pl.kernel,
    out_type=jax.ShapeDtypeStruct((NC, N_PAD, D), jnp.float32),
    mesh=_sc_mesh,
    scratch_types=[
        pltpu.VMEM((AGSZ, ACHUNK), jnp.int32),
        pltpu.VMEM((AGSZ, ACHUNK), jnp.int32),
        pltpu.VMEM((NBUF, ACHUNK, D), jnp.float32),
        pltpu.VMEM_SHARED((N_PAD, D), jnp.float32),
        pltpu.SemaphoreType.DMA,
        pltpu.SemaphoreType.DMA,
        pltpu.SemaphoreType.DMA,
        pltpu.SemaphoreType.DMA,
    ],
)
def _agg_kernel(g_hbm, src_hbm, dst_hbm, zeros_hbm, out_hbm,
                src_v, dst_v, rows, accum, s0, s1, s2, s3):
    cid = lax.axis_index("c")
    sid = lax.axis_index("s")
    wid = sid * NC + cid
    row0 = sid * ROWS_PT
    sems = (s0, s1, s2, s3)
    pltpu.sync_copy(zeros_hbm.at[pl.ds(row0, ROWS_PT)], accum.at[pl.ds(row0, ROWS_PT)])
    plsc.subcore_barrier()

    def quad(q, carry):
        i0 = 4 * q
        for k in range(4):
            i = i0 + k
            k2 = (k + 2) % 4
            pltpu.make_async_copy(g_hbm.at[src_v.at[i]], rows.at[k],
                                  sems[k]).wait()
            pltpu.async_copy(rows.at[k], accum.at[dst_v.at[i]], sems[k],
                             add=True)

            @pl.when(i >= 2)
            def _():
                pltpu.make_async_copy(rows.at[k2], accum.at[dst_v.at[i - 2]],
                                      sems[k2]).wait()

            @pl.when(i + 2 < AGSZ)
            def _():
                pltpu.async_copy(g_hbm.at[src_v.at[i + 2]], rows.at[k2],
                                 sems[k2])

        return carry

    for grp in range(NGRP):
        pltpu.sync_copy(src_hbm.at[wid, pl.ds(grp * AGSZ, AGSZ)], src_v)
        pltpu.sync_copy(dst_hbm.at[wid, pl.ds(grp * AGSZ, AGSZ)], dst_v)
        pltpu.async_copy(g_hbm.at[src_v.at[0]], rows.at[0], s0)
        pltpu.async_copy(g_hbm.at[src_v.at[1]], rows.at[1], s1)
        lax.fori_loop(0, AGSZ // 4, quad, 0)
        for t in (AGSZ - 2, AGSZ - 1):
            pltpu.make_async_copy(rows.at[t % NBUF],
                                  accum.at[dst_v.at[t]], sems[t % NBUF]).wait()

    plsc.subcore_barrier()
    pltpu.sync_copy(accum.at[pl.ds(row0, ROWS_PT)],
                    out_hbm.at[cid, pl.ds(row0, ROWS_PT)])


BR = 1024
GRID = N_PAD // BR

_row_spec = pl.BlockSpec((BR, D), lambda i: (i, 0))
_w_spec = pl.BlockSpec((D, D), lambda i: (0, 0))
_b_spec = pl.BlockSpec((1, D), lambda i: (0, 0))
_out_shape = jax.ShapeDtypeStruct((N_PAD, D), jnp.float32)


def _matmul_body(x_ref, w_ref, h_ref):
    h_ref[...] = jnp.dot(x_ref[...], w_ref[...],
                         preferred_element_type=jnp.float32)


_matmul = pl.pallas_call(
    _matmul_body,
    grid=(GRID,),
    in_specs=[_row_spec, _w_spec],
    out_specs=_row_spec,
    out_shape=_out_shape,
)


def _prep_body(p0_ref, p1_ref, h_ref, dis_ref, g_ref):
    deg = p0_ref[...][:, :1] + p1_ref[...][:, :1] + 1.0
    dis = jnp.broadcast_to(lax.rsqrt(deg), (BR, D))
    dis_ref[...] = dis
    g_ref[...] = h_ref[...] * dis


_prep = pl.pallas_call(
    _prep_body,
    grid=(GRID,),
    in_specs=[_row_spec, _row_spec, _row_spec],
    out_specs=[_row_spec, _row_spec],
    out_shape=[_out_shape, _out_shape],
)


def _mid_body(dis_ref, a0_ref, a1_ref, g_ref, b_ref, w_ref, out_ref):
    dis = dis_ref[...]
    h = (a0_ref[...] + a1_ref[...] + g_ref[...]) * dis + b_ref[...]
    h = jnp.maximum(h, 0.0)
    out_ref[...] = jnp.dot(h, w_ref[...], preferred_element_type=jnp.float32) * dis


_mid = pl.pallas_call(
    _mid_body,
    grid=(GRID,),
    in_specs=[_row_spec, _row_spec, _row_spec, _row_spec, _b_spec, _w_spec],
    out_specs=_row_spec,
    out_shape=_out_shape,
)


def _final_body(dis_ref, a0_ref, a1_ref, g_ref, b_ref, out_ref):
    out_ref[...] = ((a0_ref[...] + a1_ref[...] + g_ref[...]) * dis_ref[...]
                    + b_ref[...])


_final = pl.pallas_call(
    _final_body,
    grid=(GRID,),
    in_specs=[_row_spec, _row_spec, _row_spec, _row_spec, _b_spec],
    out_specs=_row_spec,
    out_shape=jax.ShapeDtypeStruct((N, D), jnp.float32),
)


def _pad_edges(idx):
    per_w = idx.reshape(NW, E // NW)
    pad = jnp.broadcast_to(N + (jnp.arange(PADW, dtype=jnp.int32) % (N_PAD - N)),
                           (NW, PADW))
    return jnp.concatenate([per_w, pad], axis=1)


def kernel(x, edge_index, W0, b0, W1, b1, W2, b2):
    src_flat = _pad_edges(edge_index[0].astype(jnp.int32))
    dst_flat = _pad_edges(edge_index[1].astype(jnp.int32))
    src_a = src_flat.reshape(NW, ANCH, ACHUNK)
    dst_a = dst_flat.reshape(NW, ANCH, ACHUNK)
    dst_d = dst_flat.reshape(NW, NCHUNK, CHUNK)
    x_pad = jnp.pad(x, ((0, N_PAD - N), (0, 0)))
    ones_rows = jnp.ones((CHUNK, D), jnp.float32)
    zerosD = jnp.zeros((N_PAD, D), jnp.float32)

    h0 = _matmul(x_pad, W0)
    parts = _deg_kernel(dst_d, ones_rows, zerosD)

    dis, g0 = _prep(parts[0], parts[1], h0)
    a0 = _agg_kernel(g0, src_a, dst_a, zerosD)
    g1 = _mid(dis, a0[0], a0[1], g0, b0.reshape(1, D), W1)
    a1 = _agg_kernel(g1, src_a, dst_a, zerosD)
    g2 = _mid(dis, a1[0], a1[1], g1, b1.reshape(1, D), W2)
    a2 = _agg_kernel(g2, src_a, dst_a, zerosD)
    return _final(dis, a2[0], a2[1], g2, b2.reshape(1, D))
